# baseline (speedup 1.0000x reference)
import functools

import jax
import jax.numpy as jnp
from jax import lax
from jax.experimental import pallas as pl
from jax.experimental.pallas import tpu as pltpu
from jax.experimental.pallas import tpu_sc as plsc

F32 = jnp.float32

_NC = 2
_NS = 16
_NW = _NC * _NS

_CH = 400
_SLICES = ((0, 128), (128, 128), (256, 128), (384, 16))


def _prep_body(node_ref, w1s_ref, w1r_ref, g_ref, w1g_ref, b1_ref,
               s_ref, r_ref, c_ref):
    n = node_ref[...]
    s_ref[...] = jnp.dot(n, w1s_ref[...], preferred_element_type=F32)
    r_ref[...] = jnp.dot(n, w1r_ref[...], preferred_element_type=F32)
    c_ref[...] = jnp.dot(g_ref[...], w1g_ref[...], preferred_element_type=F32) + b1_ref[...]


def _mlp_body(et_ref, gt_ref, w1et_ref, ct_ref, w2t_ref, b2t_ref, out_ref):
    x = jnp.dot(w1et_ref[...], et_ref[...], preferred_element_type=F32)
    x = x + gt_ref[...] + ct_ref[...]
    h = jnp.maximum(x, 0.0)
    out_ref[...] = jnp.dot(w2t_ref[...], h, preferred_element_type=F32) + b2t_ref[...]


def _make_gather(num_edges, latent):
    per_w = num_edges // _NW
    nchunk = per_w // _CH
    mesh = plsc.VectorSubcoreMesh(core_axis_name="c", subcore_axis_name="s")

    @functools.partial(
        pl.kernel,
        mesh=mesh,
        out_type=jax.ShapeDtypeStruct((latent, num_edges), F32),
        scratch_types=[
            pltpu.VMEM((per_w,), jnp.int32),
            pltpu.VMEM((per_w,), jnp.int32),
            pltpu.VMEM((_CH, latent), F32),
            pltpu.VMEM((_CH, latent), F32),
            pltpu.VMEM((_CH, latent), F32),
            pltpu.VMEM((_CH, latent), F32),
            pltpu.VMEM((latent, _CH + 9), F32),
            pltpu.VMEM((latent, _CH + 9), F32),
            pltpu.SemaphoreType.DMA,
            pltpu.SemaphoreType.DMA,
            pltpu.SemaphoreType.DMA,
            pltpu.SemaphoreType.DMA,
        ],
        compiler_params=pltpu.CompilerParams(
            use_tc_tiling_on_sc=False, needs_layout_passes=False),
    )
    def gather_call(s_hbm, r_hbm, src_hbm, dst_hbm, out_gt,
                    idx_s, idx_d, buf_s0, buf_r0, buf_s1, buf_r1,
                    pk0, pk1, sem0, sem1, sem_w0, sem_w1):
        wid = lax.axis_index("s") * _NC + lax.axis_index("c")
        ebase = wid * per_w
        pltpu.sync_copy(src_hbm.at[pl.ds(ebase, per_w)], idx_s)
        pltpu.sync_copy(dst_hbm.at[pl.ds(ebase, per_w)], idx_d)

        rows_lo = lax.iota(jnp.int32, 16)
        rows_hi = rows_lo + 16

        bufs = ((buf_s0, buf_r0, pk0, sem0, sem_w0),
                (buf_s1, buf_r1, pk1, sem1, sem_w1))
        pending = {}
        wpending = {}

        def fire(k):
            buf_s, buf_r, _, sem, _ = bufs[k % 2]
            hs = []
            for off, sz in _SLICES:
                lo = k * _CH + off
                hs.append(pltpu.async_copy(
                    s_hbm.at[idx_s.at[pl.ds(lo, sz)]],
                    buf_s.at[pl.ds(off, sz)], sem))
                hs.append(pltpu.async_copy(
                    r_hbm.at[idx_d.at[pl.ds(lo, sz)]],
                    buf_r.at[pl.ds(off, sz)], sem))
            pending[k % 2] = hs

        def drain_pack_write(k):
            buf_s, buf_r, pk, _, sem_w = bufs[k % 2]
            for h in pending.pop(k % 2):
                h.wait()
            if k % 2 in wpending:
                wpending.pop(k % 2).wait()

            def body(jj, carry):
                j0 = jj * 8
                for u in range(8):
                    j = j0 + u
                    col = jnp.zeros((16,), jnp.int32) + j
                    a = buf_s[j, pl.ds(0, 16)] + buf_r[j, pl.ds(0, 16)]
                    b = buf_s[j, pl.ds(16, 16)] + buf_r[j, pl.ds(16, 16)]
                    plsc.store_scatter(pk, [rows_lo, col], a)
                    plsc.store_scatter(pk, [rows_hi, col], b)
                return carry

            lax.fori_loop(0, _CH // 8, body, 0)
            cbase = ebase + k * _CH
            wpending[k % 2] = pltpu.async_copy(
                pk.at[:, pl.ds(0, _CH)], out_gt.at[:, pl.ds(cbase, _CH)], sem_w)

        fire(0)
        for k in range(1, nchunk):
            fire(k)
            drain_pack_write(k - 1)
        drain_pack_write(nchunk - 1)
        for h in wpending.values():
            h.wait()

    return gather_call


def kernel(node_attr, edge_index, edge_attr, global_attr, W1, b1, W2, b2):
    n_nodes, d_feat = node_attr.shape
    num_edges, d_edge = edge_attr.shape
    latent = W1.shape[1]
    d_out = W2.shape[1]

    src = edge_index[0].astype(jnp.int32)
    dst = edge_index[1].astype(jnp.int32)
    W1e = W1[:d_edge]
    W1s = W1[d_edge:d_edge + d_feat]
    W1r = W1[d_edge + d_feat:d_edge + 2 * d_feat]
    W1g = W1[d_edge + 2 * d_feat:]

    S, R, c = pl.pallas_call(
        _prep_body,
        out_shape=[
            jax.ShapeDtypeStruct((n_nodes, latent), F32),
            jax.ShapeDtypeStruct((n_nodes, latent), F32),
            jax.ShapeDtypeStruct((1, latent), F32),
        ],
    )(node_attr, W1s, W1r, global_attr, W1g, b1.reshape(1, latent))

    GT = _make_gather(num_edges, latent)(S, R, src, dst)

    block = 32000
    grid = num_edges // block
    outT = pl.pallas_call(
        _mlp_body,
        grid=(grid,),
        in_specs=[
            pl.BlockSpec((d_edge, block), lambda i: (0, i)),
            pl.BlockSpec((latent, block), lambda i: (0, i)),
            pl.BlockSpec((latent, d_edge), lambda i: (0, 0)),
            pl.BlockSpec((latent, 1), lambda i: (0, 0)),
            pl.BlockSpec((d_out, latent), lambda i: (0, 0)),
            pl.BlockSpec((d_out, 1), lambda i: (0, 0)),
        ],
        out_specs=pl.BlockSpec((d_out, block), lambda i: (0, i)),
        out_shape=jax.ShapeDtypeStruct((d_out, num_edges), F32),
    )(edge_attr.T, GT, W1e.T, c.T, W2.T, b2.reshape(1, d_out).T)

    return outT.T


# speedup vs baseline: 1.1767x; 1.1767x over previous
import functools

import jax
import jax.numpy as jnp
from jax import lax
from jax.experimental import pallas as pl
from jax.experimental.pallas import tpu as pltpu
from jax.experimental.pallas import tpu_sc as plsc

F32 = jnp.float32

_NC = 2
_NS = 16
_NW = _NC * _NS

_CH = 400
_SLICES = ((0, 128), (128, 128), (256, 128), (384, 16))


def _prep_body(node_ref, w1s_ref, w1r_ref, g_ref, w1g_ref, b1_ref,
               s_ref, r_ref, c_ref):
    n = node_ref[...]
    s_ref[...] = jnp.dot(n, w1s_ref[...], preferred_element_type=F32)
    r_ref[...] = jnp.dot(n, w1r_ref[...], preferred_element_type=F32)
    c_ref[...] = jnp.dot(g_ref[...], w1g_ref[...], preferred_element_type=F32) + b1_ref[...]


def _mlp_body(et_ref, gt_ref, w1et_ref, ct_ref, w2t_ref, b2t_ref, out_ref):
    x = jnp.dot(w1et_ref[...], et_ref[...], preferred_element_type=F32)
    x = x + gt_ref[...] + ct_ref[...]
    h = jnp.maximum(x, 0.0)
    out_ref[...] = jnp.dot(w2t_ref[...], h, preferred_element_type=F32) + b2t_ref[...]


def _make_gather(num_edges, latent):
    per_w = num_edges // _NW
    nchunk = per_w // _CH
    mesh = plsc.VectorSubcoreMesh(core_axis_name="c", subcore_axis_name="s")

    @functools.partial(
        pl.kernel,
        mesh=mesh,
        out_type=jax.ShapeDtypeStruct((num_edges * latent,), F32),
        scratch_types=[
            pltpu.VMEM((per_w,), jnp.int32),
            pltpu.VMEM((per_w,), jnp.int32),
            pltpu.VMEM((_CH, latent), F32),
            pltpu.VMEM((_CH, latent), F32),
            pltpu.VMEM((_CH, latent), F32),
            pltpu.VMEM((_CH, latent), F32),
            pltpu.VMEM((_CH * latent,), F32),
            pltpu.VMEM((_CH * latent,), F32),
            pltpu.SemaphoreType.DMA,
            pltpu.SemaphoreType.DMA,
            pltpu.SemaphoreType.DMA,
            pltpu.SemaphoreType.DMA,
        ],
        compiler_params=pltpu.CompilerParams(
            use_tc_tiling_on_sc=False, needs_layout_passes=False),
    )
    def gather_call(s_hbm, r_hbm, src_hbm, dst_hbm, out_gt,
                    idx_s, idx_d, buf_s0, buf_r0, buf_s1, buf_r1,
                    pk0, pk1, sem0, sem1, sem_w0, sem_w1):
        wid = lax.axis_index("s") * _NC + lax.axis_index("c")
        ebase = wid * per_w
        pltpu.sync_copy(src_hbm.at[pl.ds(ebase, per_w)], idx_s)
        pltpu.sync_copy(dst_hbm.at[pl.ds(ebase, per_w)], idx_d)

        rows_lo = lax.iota(jnp.int32, 16)
        rows_hi = rows_lo + 16

        bufs = ((buf_s0, buf_r0, pk0, sem0, sem_w0),
                (buf_s1, buf_r1, pk1, sem1, sem_w1))
        pending = {}
        wpending = {}

        def fire(k):
            buf_s, buf_r, _, sem, _ = bufs[k % 2]
            hs = []
            for off, sz in _SLICES:
                lo = k * _CH + off
                hs.append(pltpu.async_copy(
                    s_hbm.at[idx_s.at[pl.ds(lo, sz)]],
                    buf_s.at[pl.ds(off, sz)], sem))
                hs.append(pltpu.async_copy(
                    r_hbm.at[idx_d.at[pl.ds(lo, sz)]],
                    buf_r.at[pl.ds(off, sz)], sem))
            pending[k % 2] = hs

        def drain_pack_write(k):
            buf_s, buf_r, pk, _, sem_w = bufs[k % 2]
            for h in pending.pop(k % 2):
                h.wait()
            if k % 2 in wpending:
                wpending.pop(k % 2).wait()

            def body(j, carry):
                a = buf_s[j, pl.ds(0, 16)] + buf_r[j, pl.ds(0, 16)]
                b = buf_s[j, pl.ds(16, 16)] + buf_r[j, pl.ds(16, 16)]
                pk[pl.ds(32 * j, 16)] = a
                pk[pl.ds(32 * j + 16, 16)] = b
                return carry

            lax.fori_loop(0, _CH, body, 0)
            fbase = (ebase + k * _CH) * latent
            wpending[k % 2] = pltpu.async_copy(
                pk, out_gt.at[pl.ds(fbase, _CH * latent)], sem_w)

        fire(0)
        for k in range(1, nchunk):
            fire(k)
            drain_pack_write(k - 1)
        drain_pack_write(nchunk - 1)
        for h in wpending.values():
            h.wait()

    return gather_call


def kernel(node_attr, edge_index, edge_attr, global_attr, W1, b1, W2, b2):
    n_nodes, d_feat = node_attr.shape
    num_edges, d_edge = edge_attr.shape
    latent = W1.shape[1]
    d_out = W2.shape[1]

    src = edge_index[0].astype(jnp.int32)
    dst = edge_index[1].astype(jnp.int32)
    W1e = W1[:d_edge]
    W1s = W1[d_edge:d_edge + d_feat]
    W1r = W1[d_edge + d_feat:d_edge + 2 * d_feat]
    W1g = W1[d_edge + 2 * d_feat:]

    S, R, c = pl.pallas_call(
        _prep_body,
        out_shape=[
            jax.ShapeDtypeStruct((n_nodes, latent), F32),
            jax.ShapeDtypeStruct((n_nodes, latent), F32),
            jax.ShapeDtypeStruct((1, latent), F32),
        ],
    )(node_attr, W1s, W1r, global_attr, W1g, b1.reshape(1, latent))

    G1 = _make_gather(num_edges, latent)(S, R, src, dst)
    GT = jnp.swapaxes(G1.reshape(num_edges, latent), 0, 1)

    block = 32000
    grid = num_edges // block
    outT = pl.pallas_call(
        _mlp_body,
        grid=(grid,),
        in_specs=[
            pl.BlockSpec((d_edge, block), lambda i: (0, i)),
            pl.BlockSpec((latent, block), lambda i: (0, i)),
            pl.BlockSpec((latent, d_edge), lambda i: (0, 0)),
            pl.BlockSpec((latent, 1), lambda i: (0, 0)),
            pl.BlockSpec((d_out, latent), lambda i: (0, 0)),
            pl.BlockSpec((d_out, 1), lambda i: (0, 0)),
        ],
        out_specs=pl.BlockSpec((d_out, block), lambda i: (0, i)),
        out_shape=jax.ShapeDtypeStruct((d_out, num_edges), F32),
    )(edge_attr.T, GT, W1e.T, c.T, W2.T, b2.reshape(1, d_out).T)

    return outT.T
